# Initial kernel scaffold; baseline (speedup 1.0000x reference)
import jax
import jax.numpy as jnp
from jax.experimental import pallas as pl


def kernel(x, mem_tokens, g1_w, g1_b, g2_w, g2_b, pool_gamma, pool_beta, pool_w, pool_b, rel1_w, rel1_b, rel2_w, rel2_b, wq, bq, wk, bk, wv, bv, wo, bo, ln_gamma, ln_beta):
    raise NotImplementedError("write your pallas kernel here")


# speedup vs baseline: 2.3972x; 2.3972x over previous
import functools

import jax
import jax.numpy as jnp
from jax import lax
from jax.experimental import pallas as pl
from jax.experimental.pallas import tpu as pltpu

DIM = 1024
R = 3
TOPK = 256
HEADS = 8
DH = DIM // HEADS
S = 2048
KMEM = 100000
KC = 2048
NKC = 49
KPAD = KC * NKC

_INTERP = False


def _qgen_body(x_ref, w1_ref, b1_ref, w2_ref, b2_ref, pg_ref, pb_ref,
               pw_ref, pwb_ref, out_ref):
    x = x_ref[...]
    h = jax.nn.gelu(jnp.dot(x, w1_ref[0], preferred_element_type=jnp.float32)
                    + b1_ref[0])
    qf = jnp.dot(h, w2_ref[0], preferred_element_type=jnp.float32) + b2_ref[0]
    scale = 1.0 / jnp.sqrt(jnp.float32(DIM))
    w = jnp.zeros((1, S), jnp.float32)
    CH = 512
    for i in range(S // CH):
        qc = qf[i * CH:(i + 1) * CH]
        lg = lax.dot_general(qc, qf, (((1,), (1,)), ((), ())),
                             preferred_element_type=jnp.float32) * scale
        m = jnp.max(lg, axis=1, keepdims=True)
        e = jnp.exp(lg - m)
        z = jnp.sum(e, axis=1, keepdims=True)
        w = w + jnp.sum(e / z, axis=0, keepdims=True)
    pooled = jnp.dot(w, qf, preferred_element_type=jnp.float32) / jnp.float32(S)
    mu = jnp.mean(pooled, axis=1, keepdims=True)
    var = jnp.mean((pooled - mu) ** 2, axis=1, keepdims=True)
    qn = (pooled - mu) / jnp.sqrt(var + 1e-5) * pg_ref[0] + pb_ref[0]
    out_ref[0] = (jnp.dot(qn, pw_ref[0], preferred_element_type=jnp.float32)
                  + pwb_ref[0])


def _queries(x, g1_w, g1_b, g2_w, g2_b, pg, pb, pw, pwb):
    vec = pl.BlockSpec((1, 1, DIM), lambda r: (r, 0, 0))
    mat = pl.BlockSpec((1, DIM, DIM), lambda r: (r, 0, 0))
    rs = lambda t: t.reshape(R, 1, DIM)
    out = pl.pallas_call(
        _qgen_body,
        grid=(R,),
        in_specs=[pl.BlockSpec((S, DIM), lambda r: (0, 0)),
                  mat, vec, mat, vec, vec, vec, mat, vec],
        out_specs=pl.BlockSpec((1, 1, DIM), lambda r: (r, 0, 0)),
        out_shape=jax.ShapeDtypeStruct((R, 1, DIM), jnp.float32),
        interpret=_INTERP,
    )(x, g1_w, rs(g1_b), g2_w, rs(g2_b), rs(pg), rs(pb), pw, rs(pwb))
    return out.reshape(R, DIM)


def _scores_body(q_ref, mem_ref, out_ref):
    i = pl.program_id(0)
    s = lax.dot_general(q_ref[...], mem_ref[...], (((1,), (1,)), ((), ())),
                        preferred_element_type=jnp.float32)
    gi = i * KC + lax.broadcasted_iota(jnp.int32, (1, KC), 1)
    out_ref[...] = jnp.where(gi < KMEM, s, -jnp.inf)


def _scores(queries, mem):
    return pl.pallas_call(
        _scores_body,
        grid=(NKC,),
        in_specs=[
            pl.BlockSpec((R, DIM), lambda i: (0, 0)),
            pl.BlockSpec((KC, DIM), lambda i: (i, 0)),
        ],
        out_specs=pl.BlockSpec((R, KC), lambda i: (0, i)),
        out_shape=jax.ShapeDtypeStruct((R, KPAD), jnp.float32),
        interpret=_INTERP,
    )(queries, mem)


def _select_body(s_ref, tau_ref, cgt_ref):
    s = s_ref[...]
    bits = lax.bitcast_convert_type(s, jnp.uint32)
    neg = bits >= jnp.uint32(0x80000000)
    key = jnp.where(neg, ~bits, bits | jnp.uint32(0x80000000))

    def body(b, thr):
        cand = thr | (jnp.uint32(1) << (jnp.uint32(31) - b.astype(jnp.uint32)))
        cnt = jnp.sum((key >= cand).astype(jnp.int32), axis=1, keepdims=True)
        return jnp.where(cnt >= TOPK, cand, thr)

    thr = lax.fori_loop(0, 32, body, jnp.zeros((R, 1), jnp.uint32))
    cgt = jnp.sum((key > thr).astype(jnp.int32), axis=1, keepdims=True)
    obits = jnp.where(thr >= jnp.uint32(0x80000000),
                      thr ^ jnp.uint32(0x80000000), ~thr)
    tau = lax.bitcast_convert_type(obits, jnp.float32)
    tau_ref[...] = jnp.broadcast_to(tau, (R, 128))
    cgt_ref[...] = jnp.broadcast_to(cgt, (R, 128))


def _select(scores):
    return pl.pallas_call(
        _select_body,
        grid=(1,),
        in_specs=[pl.BlockSpec((R, KPAD), lambda i: (0, 0))],
        out_specs=[pl.BlockSpec((R, 128), lambda i: (0, 0)),
                   pl.BlockSpec((R, 128), lambda i: (0, 0))],
        out_shape=[jax.ShapeDtypeStruct((R, 128), jnp.float32),
                   jax.ShapeDtypeStruct((R, 128), jnp.int32)],
        interpret=_INTERP,
    )(scores)


def _fuse_body(x_ref, kv_ref, wq_ref, bq_ref, wk_ref, bk_ref, wv_ref, bv_ref,
               wo_ref, bo_ref, g_ref, b_ref, out_ref):
    x = x_ref[...]
    kv = kv_ref[...]
    q = jnp.dot(x, wq_ref[...], preferred_element_type=jnp.float32) + bq_ref[...]
    k = jnp.dot(kv, wk_ref[...], preferred_element_type=jnp.float32) + bk_ref[...]
    v = jnp.dot(kv, wv_ref[...], preferred_element_type=jnp.float32) + bv_ref[...]
    scale = 1.0 / jnp.sqrt(jnp.float32(DH))
    outs = []
    for h in range(HEADS):
        qh = q[:, h * DH:(h + 1) * DH]
        kh = k[:, h * DH:(h + 1) * DH]
        vh = v[:, h * DH:(h + 1) * DH]
        lg = lax.dot_general(qh, kh, (((1,), (1,)), ((), ())),
                             preferred_element_type=jnp.float32) * scale
        m = jnp.max(lg, axis=1, keepdims=True)
        e = jnp.exp(lg - m)
        aw = e / jnp.sum(e, axis=1, keepdims=True)
        outs.append(jnp.dot(aw, vh, preferred_element_type=jnp.float32))
    out = jnp.concatenate(outs, axis=1)
    fused = jnp.dot(out, wo_ref[...], preferred_element_type=jnp.float32) + bo_ref[...]
    y = x + fused
    mu = jnp.mean(y, axis=1, keepdims=True)
    var = jnp.mean((y - mu) ** 2, axis=1, keepdims=True)
    out_ref[...] = (y - mu) / jnp.sqrt(var + 1e-5) * g_ref[...] + b_ref[...]


def _fuse(x, kv, wq, bq, wk, bk, wv, bv, wo, bo, g, b):
    full = lambda shape: pl.BlockSpec(shape, lambda: tuple(0 for _ in shape))
    return pl.pallas_call(
        _fuse_body,
        in_specs=[
            full((S, DIM)), full((R * TOPK, DIM)),
            full((DIM, DIM)), full((1, DIM)),
            full((DIM, DIM)), full((1, DIM)),
            full((DIM, DIM)), full((1, DIM)),
            full((DIM, DIM)), full((1, DIM)),
            full((1, DIM)), full((1, DIM)),
        ],
        out_specs=full((S, DIM)),
        out_shape=jax.ShapeDtypeStruct((S, DIM), jnp.float32),
        interpret=_INTERP,
    )(x, kv, wq, bq, wk, bk, wv, bv, wo, bo, g, b)


def kernel(x, mem_tokens, g1_w, g1_b, g2_w, g2_b, pool_gamma, pool_beta,
           pool_w, pool_b, rel1_w, rel1_b, rel2_w, rel2_b, wq, bq, wk, bk,
           wv, bv, wo, bo, ln_gamma, ln_beta):
    x2 = x[0]
    row = lambda t: t.reshape(1, -1)
    queries = _queries(x2, g1_w, g1_b, g2_w, g2_b, pool_gamma, pool_beta,
                       pool_w, pool_b)
    scores = _scores(queries, mem_tokens)
    _, idx = lax.top_k(scores[:, :KMEM], TOPK)
    kv = mem_tokens[idx.reshape(-1)]
    out = _fuse(x2, kv, wq, row(bq), wk, row(bk), wv, row(bv), wo, row(bo),
                row(ln_gamma), row(ln_beta))
    return out[None]


# speedup vs baseline: 3.7145x; 1.5495x over previous
import functools

import jax
import jax.numpy as jnp
from jax import lax
from jax.experimental import pallas as pl
from jax.experimental.pallas import tpu as pltpu
from jax.experimental.pallas import tpu_sc as plsc

DIM = 1024
R = 3
TOPK = 256
HEADS = 8
DH = DIM // HEADS
S = 2048
KMEM = 100000
KC = 2048
NKC = 49
KPAD = KC * NKC

_INTERP = False


def _qgen_body(x_ref, w1_ref, b1_ref, w2_ref, b2_ref, pg_ref, pb_ref,
               pw_ref, pwb_ref, out_ref):
    x = x_ref[...]
    h = jax.nn.gelu(jnp.dot(x, w1_ref[0], preferred_element_type=jnp.float32)
                    + b1_ref[0])
    qf = jnp.dot(h, w2_ref[0], preferred_element_type=jnp.float32) + b2_ref[0]
    scale = 1.0 / jnp.sqrt(jnp.float32(DIM))
    w = jnp.zeros((1, S), jnp.float32)
    CH = 512
    for i in range(S // CH):
        qc = qf[i * CH:(i + 1) * CH]
        lg = lax.dot_general(qc, qf, (((1,), (1,)), ((), ())),
                             preferred_element_type=jnp.float32) * scale
        m = jnp.max(lg, axis=1, keepdims=True)
        e = jnp.exp(lg - m)
        z = jnp.sum(e, axis=1, keepdims=True)
        w = w + jnp.sum(e / z, axis=0, keepdims=True)
    pooled = jnp.dot(w, qf, preferred_element_type=jnp.float32) / jnp.float32(S)
    mu = jnp.mean(pooled, axis=1, keepdims=True)
    var = jnp.mean((pooled - mu) ** 2, axis=1, keepdims=True)
    qn = (pooled - mu) / jnp.sqrt(var + 1e-5) * pg_ref[0] + pb_ref[0]
    out_ref[0] = (jnp.dot(qn, pw_ref[0], preferred_element_type=jnp.float32)
                  + pwb_ref[0])


def _queries(x, g1_w, g1_b, g2_w, g2_b, pg, pb, pw, pwb):
    vec = pl.BlockSpec((1, 1, DIM), lambda r: (r, 0, 0))
    mat = pl.BlockSpec((1, DIM, DIM), lambda r: (r, 0, 0))
    rs = lambda t: t.reshape(R, 1, DIM)
    out = pl.pallas_call(
        _qgen_body,
        grid=(R,),
        in_specs=[pl.BlockSpec((S, DIM), lambda r: (0, 0)),
                  mat, vec, mat, vec, vec, vec, mat, vec],
        out_specs=pl.BlockSpec((1, 1, DIM), lambda r: (r, 0, 0)),
        out_shape=jax.ShapeDtypeStruct((R, 1, DIM), jnp.float32),
        interpret=_INTERP,
    )(x, g1_w, rs(g1_b), g2_w, rs(g2_b), rs(pg), rs(pb), pw, rs(pwb))
    return out.reshape(R, DIM)


def _scores_body(q_ref, mem_ref, out_ref):
    i = pl.program_id(0)
    s = lax.dot_general(q_ref[...], mem_ref[...], (((1,), (1,)), ((), ())),
                        preferred_element_type=jnp.float32)
    gi = i * KC + lax.broadcasted_iota(jnp.int32, (1, KC), 1)
    out_ref[...] = jnp.where(gi < KMEM, s, -jnp.inf)


def _scores(queries, mem):
    return pl.pallas_call(
        _scores_body,
        grid=(NKC,),
        in_specs=[
            pl.BlockSpec((R, DIM), lambda i: (0, 0)),
            pl.BlockSpec((KC, DIM), lambda i: (i, 0)),
        ],
        out_specs=pl.BlockSpec((R, KC), lambda i: (0, i)),
        out_shape=jax.ShapeDtypeStruct((R, KPAD), jnp.float32),
        interpret=_INTERP,
    )(queries, mem)


def _select_body(s_ref, tau_ref, cgt_ref):
    s = s_ref[...]
    bits = lax.bitcast_convert_type(s, jnp.uint32)
    neg = bits >= jnp.uint32(0x80000000)
    key = jnp.where(neg, ~bits, bits | jnp.uint32(0x80000000))

    def body(b, thr):
        cand = thr | (jnp.uint32(1) << (jnp.uint32(31) - b.astype(jnp.uint32)))
        cnt = jnp.sum((key >= cand).astype(jnp.int32), axis=1, keepdims=True)
        return jnp.where(cnt >= TOPK, cand, thr)

    thr = lax.fori_loop(0, 32, body, jnp.zeros((R, 1), jnp.uint32))
    cgt = jnp.sum((key > thr).astype(jnp.int32), axis=1, keepdims=True)
    obits = jnp.where(thr >= jnp.uint32(0x80000000),
                      thr ^ jnp.uint32(0x80000000), ~thr)
    tau = lax.bitcast_convert_type(obits, jnp.float32)
    tau_ref[...] = jnp.broadcast_to(tau, (R, 128))
    cgt_ref[...] = jnp.broadcast_to(cgt, (R, 128))


def _select(scores):
    return pl.pallas_call(
        _select_body,
        grid=(1,),
        in_specs=[pl.BlockSpec((R, KPAD), lambda i: (0, 0))],
        out_specs=[pl.BlockSpec((R, 128), lambda i: (0, 0)),
                   pl.BlockSpec((R, 128), lambda i: (0, 0))],
        out_shape=[jax.ShapeDtypeStruct((R, 128), jnp.float32),
                   jax.ShapeDtypeStruct((R, 128), jnp.int32)],
        interpret=_INTERP,
    )(scores)


NS = 16
SHARD = KPAD // NS
NCHUNK = SHARD // 16
CAND = 272


def _sc_retrieve(scores_flat, tau_flat, cgt_flat, mem):
    mesh = plsc.VectorSubcoreMesh(core_axis_name="c", subcore_axis_name="s")

    @functools.partial(
        pl.kernel, mesh=mesh,
        out_type=jax.ShapeDtypeStruct((R * TOPK, DIM), jnp.float32),
        scratch_types=[
            pltpu.VMEM((SHARD,), jnp.float32),
            pltpu.VMEM((16,), jnp.float32),
            pltpu.VMEM((CAND,), jnp.int32),
            pltpu.VMEM((CAND,), jnp.int32),
            pltpu.VMEM((16,), jnp.int32),
            pltpu.VMEM((NS * 16,), jnp.int32),
            pltpu.VMEM((NS * 16,), jnp.int32),
            pltpu.VMEM((CAND,), jnp.int32),
            pltpu.VMEM((TOPK,), jnp.int32),
            pltpu.VMEM((16,), jnp.int32),
            pltpu.VMEM((16, DIM), jnp.float32),
            pltpu.VMEM_SHARED((NS * CAND,), jnp.int32),
            pltpu.VMEM_SHARED((NS * CAND,), jnp.int32),
            pltpu.VMEM_SHARED((NS * 16,), jnp.int32),
            pltpu.VMEM_SHARED((NS * 16,), jnp.int32),
            pltpu.VMEM_SHARED((TOPK,), jnp.int32),
            pltpu.SemaphoreType.DMA,
        ],
        compiler_params=pltpu.CompilerParams(needs_layout_passes=False),
    )
    def body(scores_hbm, tau_hbm, cgt_hbm, mem_hbm, kv_hbm, sbuf, tauv,
             cand_gt, cand_eq, cntst, cntg_all, cntq_all, tmpbuf, idx_vmem,
             idx16, rowbuf, gt_sh, eq_sh, cntg_sh, cntq_sh, idx_sh, sem):
        del cgt_hbm
        cid = lax.axis_index("c")
        sid = lax.axis_index("s")
        iota = lax.iota(jnp.int32, 16)
        zero = jnp.zeros((16,), jnp.int32)
        one = jnp.ones((16,), jnp.int32)

        for r in range(R):
            core = r % 2

            @pl.when(cid == core)
            def _do_r(r=r):
                base = sid * SHARD
                pltpu.sync_copy(
                    scores_hbm.at[pl.ds(r * KPAD + base, SHARD)], sbuf)
                pltpu.sync_copy(tau_hbm.at[pl.ds(r * 128, 16)], tauv)
                tau = tauv[...]

                def scan_body(i, carry):
                    og, oq = carry
                    v = sbuf[pl.ds(i * 16, 16)]
                    gidx = base + i * 16 + iota
                    gt = v > tau
                    eq = v == tau
                    pos_g = og + plsc.cumsum(jnp.where(gt, one, zero)) - 1
                    gt2 = gt & (pos_g < TOPK)
                    plsc.store_scatter(cand_gt,
                                       [jnp.clip(pos_g, 0, CAND - 1)],
                                       gidx, mask=gt2)
                    pos_q = oq + plsc.cumsum(jnp.where(eq, one, zero)) - 1
                    eq2 = eq & (pos_q < TOPK)
                    plsc.store_scatter(cand_eq,
                                       [jnp.clip(pos_q, 0, CAND - 1)],
                                       gidx, mask=eq2)
                    return (og + plsc.all_reduce_population_count(gt2),
                            oq + plsc.all_reduce_population_count(eq2))

                og, oq = lax.fori_loop(0, NCHUNK, scan_body, (zero, zero))
                cntst[...] = og
                pltpu.sync_copy(cntst, cntg_sh.at[pl.ds(sid * 16, 16)])
                cntst[...] = oq
                pltpu.sync_copy(cntst, cntq_sh.at[pl.ds(sid * 16, 16)])
                pltpu.sync_copy(cand_gt, gt_sh.at[pl.ds(sid * CAND, CAND)])
                pltpu.sync_copy(cand_eq, eq_sh.at[pl.ds(sid * CAND, CAND)])
                plsc.subcore_barrier()

                @pl.when(sid == 0)
                def _assemble():
                    pltpu.sync_copy(cntg_sh, cntg_all)
                    pltpu.sync_copy(cntq_sh, cntq_all)

                    def merge(cnt_all, src_sh, off):
                        for j in range(NS):
                            cj = cnt_all[pl.ds(j * 16, 16)]
                            pltpu.sync_copy(
                                src_sh.at[pl.ds(j * CAND, CAND)], tmpbuf)

                            def abody(c, off, cj=cj):
                                v = tmpbuf[pl.ds(c * 16, 16)]
                                valid = (c * 16 + iota) < cj
                                pos = off + plsc.cumsum(
                                    jnp.where(valid, one, zero)) - 1
                                keep = valid & (pos < TOPK)
                                plsc.store_scatter(
                                    idx_vmem, [jnp.clip(pos, 0, TOPK - 1)],
                                    v, mask=keep)
                                return off + plsc.all_reduce_population_count(keep)

                            off = lax.fori_loop(0, CAND // 16, abody, off)
                        return off

                    off = merge(cntg_all, gt_sh, zero)
                    merge(cntq_all, eq_sh, off)
                    pltpu.sync_copy(idx_vmem, idx_sh)

                plsc.subcore_barrier()

                pltpu.sync_copy(idx_sh.at[pl.ds(sid * 16, 16)], idx16)
                idx16[...] = jnp.clip(idx16[...], 0, KMEM - 1)
                pltpu.async_copy(mem_hbm.at[idx16], rowbuf, sem).wait()
                pltpu.sync_copy(rowbuf,
                                kv_hbm.at[pl.ds(r * TOPK + sid * 16, 16)])

    return body(scores_flat, tau_flat, cgt_flat, mem)


def _fuse_body(x_ref, kv_ref, wq_ref, bq_ref, wk_ref, bk_ref, wv_ref, bv_ref,
               wo_ref, bo_ref, g_ref, b_ref, out_ref):
    x = x_ref[...]
    kv = kv_ref[...]
    q = jnp.dot(x, wq_ref[...], preferred_element_type=jnp.float32) + bq_ref[...]
    k = jnp.dot(kv, wk_ref[...], preferred_element_type=jnp.float32) + bk_ref[...]
    v = jnp.dot(kv, wv_ref[...], preferred_element_type=jnp.float32) + bv_ref[...]
    scale = 1.0 / jnp.sqrt(jnp.float32(DH))
    outs = []
    for h in range(HEADS):
        qh = q[:, h * DH:(h + 1) * DH]
        kh = k[:, h * DH:(h + 1) * DH]
        vh = v[:, h * DH:(h + 1) * DH]
        lg = lax.dot_general(qh, kh, (((1,), (1,)), ((), ())),
                             preferred_element_type=jnp.float32) * scale
        m = jnp.max(lg, axis=1, keepdims=True)
        e = jnp.exp(lg - m)
        aw = e / jnp.sum(e, axis=1, keepdims=True)
        outs.append(jnp.dot(aw, vh, preferred_element_type=jnp.float32))
    out = jnp.concatenate(outs, axis=1)
    fused = jnp.dot(out, wo_ref[...], preferred_element_type=jnp.float32) + bo_ref[...]
    y = x + fused
    mu = jnp.mean(y, axis=1, keepdims=True)
    var = jnp.mean((y - mu) ** 2, axis=1, keepdims=True)
    out_ref[...] = (y - mu) / jnp.sqrt(var + 1e-5) * g_ref[...] + b_ref[...]


def _fuse(x, kv, wq, bq, wk, bk, wv, bv, wo, bo, g, b):
    full = lambda shape: pl.BlockSpec(shape, lambda: tuple(0 for _ in shape))
    return pl.pallas_call(
        _fuse_body,
        in_specs=[
            full((S, DIM)), full((R * TOPK, DIM)),
            full((DIM, DIM)), full((1, DIM)),
            full((DIM, DIM)), full((1, DIM)),
            full((DIM, DIM)), full((1, DIM)),
            full((DIM, DIM)), full((1, DIM)),
            full((1, DIM)), full((1, DIM)),
        ],
        out_specs=full((S, DIM)),
        out_shape=jax.ShapeDtypeStruct((S, DIM), jnp.float32),
        interpret=_INTERP,
    )(x, kv, wq, bq, wk, bk, wv, bv, wo, bo, g, b)


def kernel(x, mem_tokens, g1_w, g1_b, g2_w, g2_b, pool_gamma, pool_beta,
           pool_w, pool_b, rel1_w, rel1_b, rel2_w, rel2_b, wq, bq, wk, bk,
           wv, bv, wo, bo, ln_gamma, ln_beta):
    x2 = x[0]
    row = lambda t: t.reshape(1, -1)
    queries = _queries(x2, g1_w, g1_b, g2_w, g2_b, pool_gamma, pool_beta,
                       pool_w, pool_b)
    scores = _scores(queries, mem_tokens)
    tau, cgt = _select(scores)
    kv = _sc_retrieve(scores.reshape(-1), tau.reshape(-1), cgt.reshape(-1),
                      mem_tokens)
    out = _fuse(x2, kv, wq, row(bq), wk, row(bk), wv, row(bv), wo, row(bo),
                row(ln_gamma), row(ln_beta))
    return out[None]


# speedup vs baseline: 3.7194x; 1.0013x over previous
import functools

import jax
import jax.numpy as jnp
from jax import lax
from jax.experimental import pallas as pl
from jax.experimental.pallas import tpu as pltpu
from jax.experimental.pallas import tpu_sc as plsc

DIM = 1024
R = 3
TOPK = 256
HEADS = 8
DH = DIM // HEADS
S = 2048
KMEM = 100000
KC = 2048
NKC = 49
KPAD = KC * NKC

_INTERP = False


def _qgen_body(x_ref, w1_ref, b1_ref, w2_ref, b2_ref, pg_ref, pb_ref,
               pw_ref, pwb_ref, out_ref):
    bf = jnp.bfloat16
    x = x_ref[...]
    h = jax.nn.gelu(jnp.dot(x.astype(bf), w1_ref[0].astype(bf),
                            preferred_element_type=jnp.float32) + b1_ref[0])
    qf = jnp.dot(h.astype(bf), w2_ref[0].astype(bf),
                 preferred_element_type=jnp.float32) + b2_ref[0]
    qfb = qf.astype(bf)
    scale = 1.0 / jnp.sqrt(jnp.float32(DIM))
    w = jnp.zeros((1, S), jnp.float32)
    CH = 512
    for i in range(S // CH):
        qc = qfb[i * CH:(i + 1) * CH]
        lg = lax.dot_general(qc, qfb, (((1,), (1,)), ((), ())),
                             preferred_element_type=jnp.float32) * scale
        m = jnp.max(lg, axis=1, keepdims=True)
        e = jnp.exp(lg - m)
        z = jnp.sum(e, axis=1, keepdims=True)
        w = w + jnp.sum(e / z, axis=0, keepdims=True)
    pooled = jnp.dot(w, qf, preferred_element_type=jnp.float32) / jnp.float32(S)
    mu = jnp.mean(pooled, axis=1, keepdims=True)
    var = jnp.mean((pooled - mu) ** 2, axis=1, keepdims=True)
    qn = (pooled - mu) / jnp.sqrt(var + 1e-5) * pg_ref[0] + pb_ref[0]
    out_ref[0] = (jnp.dot(qn, pw_ref[0], preferred_element_type=jnp.float32)
                  + pwb_ref[0])


def _queries(x, g1_w, g1_b, g2_w, g2_b, pg, pb, pw, pwb):
    vec = pl.BlockSpec((1, 1, DIM), lambda r: (r, 0, 0))
    mat = pl.BlockSpec((1, DIM, DIM), lambda r: (r, 0, 0))
    rs = lambda t: t.reshape(R, 1, DIM)
    out = pl.pallas_call(
        _qgen_body,
        grid=(R,),
        in_specs=[pl.BlockSpec((S, DIM), lambda r: (0, 0)),
                  mat, vec, mat, vec, vec, vec, mat, vec],
        out_specs=pl.BlockSpec((1, 1, DIM), lambda r: (r, 0, 0)),
        out_shape=jax.ShapeDtypeStruct((R, 1, DIM), jnp.float32),
        interpret=_INTERP,
    )(x, g1_w, rs(g1_b), g2_w, rs(g2_b), rs(pg), rs(pb), pw, rs(pwb))
    return out.reshape(R, DIM)


def _scores_body(q_ref, mem_ref, out_ref):
    i = pl.program_id(0)
    s = lax.dot_general(q_ref[...], mem_ref[...], (((1,), (1,)), ((), ())),
                        preferred_element_type=jnp.float32)
    gi = i * KC + lax.broadcasted_iota(jnp.int32, (1, KC), 1)
    out_ref[...] = jnp.where(gi < KMEM, s, -jnp.inf)


def _scores(queries, mem):
    return pl.pallas_call(
        _scores_body,
        grid=(NKC,),
        in_specs=[
            pl.BlockSpec((R, DIM), lambda i: (0, 0)),
            pl.BlockSpec((KC, DIM), lambda i: (i, 0)),
        ],
        out_specs=pl.BlockSpec((R, KC), lambda i: (0, i)),
        out_shape=jax.ShapeDtypeStruct((R, KPAD), jnp.float32),
        interpret=_INTERP,
    )(queries, mem)


def _select_body(s_ref, tau_ref, cgt_ref):
    s = s_ref[...]
    bits = lax.bitcast_convert_type(s, jnp.uint32)
    neg = bits >= jnp.uint32(0x80000000)
    key = jnp.where(neg, ~bits, bits | jnp.uint32(0x80000000))

    def body(b, thr):
        cand = thr | (jnp.uint32(1) << (jnp.uint32(31) - b.astype(jnp.uint32)))
        cnt = jnp.sum((key >= cand).astype(jnp.int32), axis=1, keepdims=True)
        return jnp.where(cnt >= TOPK, cand, thr)

    thr = lax.fori_loop(0, 32, body, jnp.zeros((R, 1), jnp.uint32))
    cgt = jnp.sum((key > thr).astype(jnp.int32), axis=1, keepdims=True)
    obits = jnp.where(thr >= jnp.uint32(0x80000000),
                      thr ^ jnp.uint32(0x80000000), ~thr)
    tau = lax.bitcast_convert_type(obits, jnp.float32)
    tau_ref[...] = jnp.broadcast_to(tau, (R, 128))
    cgt_ref[...] = jnp.broadcast_to(cgt, (R, 128))


def _select(scores):
    return pl.pallas_call(
        _select_body,
        grid=(1,),
        in_specs=[pl.BlockSpec((R, KPAD), lambda i: (0, 0))],
        out_specs=[pl.BlockSpec((R, 128), lambda i: (0, 0)),
                   pl.BlockSpec((R, 128), lambda i: (0, 0))],
        out_shape=[jax.ShapeDtypeStruct((R, 128), jnp.float32),
                   jax.ShapeDtypeStruct((R, 128), jnp.int32)],
        interpret=_INTERP,
    )(scores)


NS = 16
SHARD = KPAD // NS
NCHUNK = SHARD // 16
CAND = 272


def _sc_retrieve(scores_flat, tau_flat, cgt_flat, mem):
    mesh = plsc.VectorSubcoreMesh(core_axis_name="c", subcore_axis_name="s")

    @functools.partial(
        pl.kernel, mesh=mesh,
        out_type=jax.ShapeDtypeStruct((R * TOPK, DIM), jnp.float32),
        scratch_types=[
            pltpu.VMEM((SHARD,), jnp.float32),
            pltpu.VMEM((16,), jnp.float32),
            pltpu.VMEM((CAND,), jnp.int32),
            pltpu.VMEM((CAND,), jnp.int32),
            pltpu.VMEM((16,), jnp.int32),
            pltpu.VMEM((NS * 16,), jnp.int32),
            pltpu.VMEM((NS * 16,), jnp.int32),
            pltpu.VMEM((CAND,), jnp.int32),
            pltpu.VMEM((TOPK,), jnp.int32),
            pltpu.VMEM((16,), jnp.int32),
            pltpu.VMEM((16, DIM), jnp.float32),
            pltpu.VMEM_SHARED((NS * CAND,), jnp.int32),
            pltpu.VMEM_SHARED((NS * CAND,), jnp.int32),
            pltpu.VMEM_SHARED((NS * 16,), jnp.int32),
            pltpu.VMEM_SHARED((NS * 16,), jnp.int32),
            pltpu.VMEM_SHARED((TOPK,), jnp.int32),
            pltpu.SemaphoreType.DMA,
        ],
        compiler_params=pltpu.CompilerParams(needs_layout_passes=False),
    )
    def body(scores_hbm, tau_hbm, cgt_hbm, mem_hbm, kv_hbm, sbuf, tauv,
             cand_gt, cand_eq, cntst, cntg_all, cntq_all, tmpbuf, idx_vmem,
             idx16, rowbuf, gt_sh, eq_sh, cntg_sh, cntq_sh, idx_sh, sem):
        del cgt_hbm
        cid = lax.axis_index("c")
        sid = lax.axis_index("s")
        iota = lax.iota(jnp.int32, 16)
        zero = jnp.zeros((16,), jnp.int32)
        one = jnp.ones((16,), jnp.int32)

        for r in range(R):
            core = r % 2

            @pl.when(cid == core)
            def _do_r(r=r):
                base = sid * SHARD
                pltpu.sync_copy(
                    scores_hbm.at[pl.ds(r * KPAD + base, SHARD)], sbuf)
                pltpu.sync_copy(tau_hbm.at[pl.ds(r * 128, 16)], tauv)
                tau = tauv[...]

                def scan_body(i, carry):
                    og, oq = carry
                    v = sbuf[pl.ds(i * 16, 16)]
                    gidx = base + i * 16 + iota
                    gt = v > tau
                    eq = v == tau
                    pos_g = og + plsc.cumsum(jnp.where(gt, one, zero)) - 1
                    gt2 = gt & (pos_g < TOPK)
                    plsc.store_scatter(cand_gt,
                                       [jnp.clip(pos_g, 0, CAND - 1)],
                                       gidx, mask=gt2)
                    pos_q = oq + plsc.cumsum(jnp.where(eq, one, zero)) - 1
                    eq2 = eq & (pos_q < TOPK)
                    plsc.store_scatter(cand_eq,
                                       [jnp.clip(pos_q, 0, CAND - 1)],
                                       gidx, mask=eq2)
                    return (og + plsc.all_reduce_population_count(gt2),
                            oq + plsc.all_reduce_population_count(eq2))

                og, oq = lax.fori_loop(0, NCHUNK, scan_body, (zero, zero))
                cntst[...] = og
                pltpu.sync_copy(cntst, cntg_sh.at[pl.ds(sid * 16, 16)])
                cntst[...] = oq
                pltpu.sync_copy(cntst, cntq_sh.at[pl.ds(sid * 16, 16)])
                pltpu.sync_copy(cand_gt, gt_sh.at[pl.ds(sid * CAND, CAND)])
                pltpu.sync_copy(cand_eq, eq_sh.at[pl.ds(sid * CAND, CAND)])
                plsc.subcore_barrier()

                @pl.when(sid == 0)
                def _assemble():
                    pltpu.sync_copy(cntg_sh, cntg_all)
                    pltpu.sync_copy(cntq_sh, cntq_all)

                    def merge(cnt_all, src_sh, off):
                        for j in range(NS):
                            cj = cnt_all[pl.ds(j * 16, 16)]
                            pltpu.sync_copy(
                                src_sh.at[pl.ds(j * CAND, CAND)], tmpbuf)

                            def abody(c, off, cj=cj):
                                v = tmpbuf[pl.ds(c * 16, 16)]
                                valid = (c * 16 + iota) < cj
                                pos = off + plsc.cumsum(
                                    jnp.where(valid, one, zero)) - 1
                                keep = valid & (pos < TOPK)
                                plsc.store_scatter(
                                    idx_vmem, [jnp.clip(pos, 0, TOPK - 1)],
                                    v, mask=keep)
                                return off + plsc.all_reduce_population_count(keep)

                            off = lax.fori_loop(0, CAND // 16, abody, off)
                        return off

                    off = merge(cntg_all, gt_sh, zero)
                    merge(cntq_all, eq_sh, off)
                    pltpu.sync_copy(idx_vmem, idx_sh)

                plsc.subcore_barrier()

                pltpu.sync_copy(idx_sh.at[pl.ds(sid * 16, 16)], idx16)
                idx16[...] = jnp.clip(idx16[...], 0, KMEM - 1)
                pltpu.async_copy(mem_hbm.at[idx16], rowbuf, sem).wait()
                pltpu.sync_copy(rowbuf,
                                kv_hbm.at[pl.ds(r * TOPK + sid * 16, 16)])

    return body(scores_flat, tau_flat, cgt_flat, mem)


def _fuse_body(x_ref, kv_ref, wq_ref, bq_ref, wk_ref, bk_ref, wv_ref, bv_ref,
               wo_ref, bo_ref, g_ref, b_ref, out_ref):
    bf = jnp.bfloat16
    x = x_ref[...]
    kv = kv_ref[...].astype(bf)
    xb = x.astype(bf)
    q = (jnp.dot(xb, wq_ref[...].astype(bf),
                 preferred_element_type=jnp.float32) + bq_ref[...]).astype(bf)
    k = (jnp.dot(kv, wk_ref[...].astype(bf),
                 preferred_element_type=jnp.float32) + bk_ref[...]).astype(bf)
    v = (jnp.dot(kv, wv_ref[...].astype(bf),
                 preferred_element_type=jnp.float32) + bv_ref[...]).astype(bf)
    scale = 1.0 / jnp.sqrt(jnp.float32(DH))
    outs = []
    for h in range(HEADS):
        qh = q[:, h * DH:(h + 1) * DH]
        kh = k[:, h * DH:(h + 1) * DH]
        vh = v[:, h * DH:(h + 1) * DH]
        lg = lax.dot_general(qh, kh, (((1,), (1,)), ((), ())),
                             preferred_element_type=jnp.float32) * scale
        m = jnp.max(lg, axis=1, keepdims=True)
        e = jnp.exp(lg - m)
        aw = (e / jnp.sum(e, axis=1, keepdims=True)).astype(jnp.bfloat16)
        outs.append(jnp.dot(aw, vh, preferred_element_type=jnp.float32))
    out = jnp.concatenate(outs, axis=1).astype(jnp.bfloat16)
    fused = jnp.dot(out, wo_ref[...].astype(jnp.bfloat16),
                    preferred_element_type=jnp.float32) + bo_ref[...]
    y = x + fused
    mu = jnp.mean(y, axis=1, keepdims=True)
    var = jnp.mean((y - mu) ** 2, axis=1, keepdims=True)
    out_ref[...] = (y - mu) / jnp.sqrt(var + 1e-5) * g_ref[...] + b_ref[...]


def _fuse(x, kv, wq, bq, wk, bk, wv, bv, wo, bo, g, b):
    full = lambda shape: pl.BlockSpec(shape, lambda: tuple(0 for _ in shape))
    return pl.pallas_call(
        _fuse_body,
        in_specs=[
            full((S, DIM)), full((R * TOPK, DIM)),
            full((DIM, DIM)), full((1, DIM)),
            full((DIM, DIM)), full((1, DIM)),
            full((DIM, DIM)), full((1, DIM)),
            full((DIM, DIM)), full((1, DIM)),
            full((1, DIM)), full((1, DIM)),
        ],
        out_specs=full((S, DIM)),
        out_shape=jax.ShapeDtypeStruct((S, DIM), jnp.float32),
        interpret=_INTERP,
    )(x, kv, wq, bq, wk, bk, wv, bv, wo, bo, g, b)


def kernel(x, mem_tokens, g1_w, g1_b, g2_w, g2_b, pool_gamma, pool_beta,
           pool_w, pool_b, rel1_w, rel1_b, rel2_w, rel2_b, wq, bq, wk, bk,
           wv, bv, wo, bo, ln_gamma, ln_beta):
    x2 = x[0]
    row = lambda t: t.reshape(1, -1)
    queries = _queries(x2, g1_w, g1_b, g2_w, g2_b, pool_gamma, pool_beta,
                       pool_w, pool_b)
    scores = _scores(queries, mem_tokens)
    tau, cgt = _select(scores)
    kv = _sc_retrieve(scores.reshape(-1), tau.reshape(-1), cgt.reshape(-1),
                      mem_tokens)
    out = _fuse(x2, kv, wq, row(bq), wk, row(bk), wv, row(bv), wo, row(bo),
                row(ln_gamma), row(ln_beta))
    return out[None]


# speedup vs baseline: 3.7503x; 1.0083x over previous
import functools

import jax
import jax.numpy as jnp
from jax import lax
from jax.experimental import pallas as pl
from jax.experimental.pallas import tpu as pltpu
from jax.experimental.pallas import tpu_sc as plsc

DIM = 1024
R = 3
TOPK = 256
HEADS = 8
DH = DIM // HEADS
S = 2048
KMEM = 100000
KC = 2048
NKC = 49
KPAD = KC * NKC

_INTERP = False


def _qgen_body(x_ref, w1_ref, b1_ref, w2_ref, b2_ref, pg_ref, pb_ref,
               pw_ref, pwb_ref, out_ref):
    bf = jnp.bfloat16
    x = x_ref[...]
    h = jax.nn.gelu(jnp.dot(x.astype(bf), w1_ref[0].astype(bf),
                            preferred_element_type=jnp.float32) + b1_ref[0])
    qf = jnp.dot(h.astype(bf), w2_ref[0].astype(bf),
                 preferred_element_type=jnp.float32) + b2_ref[0]
    qfb = qf.astype(bf)
    scale = 1.0 / jnp.sqrt(jnp.float32(DIM))
    w = jnp.zeros((1, S), jnp.float32)
    CH = 512
    for i in range(S // CH):
        qc = qfb[i * CH:(i + 1) * CH]
        lg = lax.dot_general(qc, qfb, (((1,), (1,)), ((), ())),
                             preferred_element_type=jnp.float32) * scale
        m = jnp.max(lg, axis=1, keepdims=True)
        e = jnp.exp(lg - m)
        z = jnp.sum(e, axis=1, keepdims=True)
        w = w + jnp.sum(e / z, axis=0, keepdims=True)
    pooled = jnp.dot(w, qf, preferred_element_type=jnp.float32) / jnp.float32(S)
    mu = jnp.mean(pooled, axis=1, keepdims=True)
    var = jnp.mean((pooled - mu) ** 2, axis=1, keepdims=True)
    qn = (pooled - mu) / jnp.sqrt(var + 1e-5) * pg_ref[0] + pb_ref[0]
    out_ref[0] = (jnp.dot(qn, pw_ref[0], preferred_element_type=jnp.float32)
                  + pwb_ref[0])


def _queries(x, g1_w, g1_b, g2_w, g2_b, pg, pb, pw, pwb):
    vec = pl.BlockSpec((1, 1, DIM), lambda r: (r, 0, 0))
    mat = pl.BlockSpec((1, DIM, DIM), lambda r: (r, 0, 0))
    rs = lambda t: t.reshape(R, 1, DIM)
    out = pl.pallas_call(
        _qgen_body,
        grid=(R,),
        in_specs=[pl.BlockSpec((S, DIM), lambda r: (0, 0)),
                  mat, vec, mat, vec, vec, vec, mat, vec],
        out_specs=pl.BlockSpec((1, 1, DIM), lambda r: (r, 0, 0)),
        out_shape=jax.ShapeDtypeStruct((R, 1, DIM), jnp.float32),
        interpret=_INTERP,
    )(x, g1_w, rs(g1_b), g2_w, rs(g2_b), rs(pg), rs(pb), pw, rs(pwb))
    return out.reshape(R, DIM)


def _scores_body(q_ref, mem_ref, out_ref):
    i = pl.program_id(0)
    s = lax.dot_general(q_ref[...], mem_ref[...], (((1,), (1,)), ((), ())),
                        preferred_element_type=jnp.float32)
    gi = i * KC + lax.broadcasted_iota(jnp.int32, (1, KC), 1)
    out_ref[...] = jnp.where(gi < KMEM, s, -jnp.inf)


def _scores(queries, mem):
    return pl.pallas_call(
        _scores_body,
        grid=(NKC,),
        in_specs=[
            pl.BlockSpec((R, DIM), lambda i: (0, 0)),
            pl.BlockSpec((KC, DIM), lambda i: (i, 0)),
        ],
        out_specs=pl.BlockSpec((R, KC), lambda i: (0, i)),
        out_shape=jax.ShapeDtypeStruct((R, KPAD), jnp.float32),
        interpret=_INTERP,
    )(queries, mem)


def _select_body(s_ref, tau_ref, cgt_ref):
    s = s_ref[...]
    bits = lax.bitcast_convert_type(s, jnp.uint32)
    neg = bits >= jnp.uint32(0x80000000)
    key = jnp.where(neg, ~bits, bits | jnp.uint32(0x80000000))

    def body(b, thr):
        cand = thr | (jnp.uint32(1) << (jnp.uint32(31) - b.astype(jnp.uint32)))
        cnt = jnp.sum((key >= cand).astype(jnp.int32), axis=1, keepdims=True)
        return jnp.where(cnt >= TOPK, cand, thr)

    thr = lax.fori_loop(0, 32, body, jnp.zeros((R, 1), jnp.uint32))
    cgt = jnp.sum((key > thr).astype(jnp.int32), axis=1, keepdims=True)
    obits = jnp.where(thr >= jnp.uint32(0x80000000),
                      thr ^ jnp.uint32(0x80000000), ~thr)
    tau = lax.bitcast_convert_type(obits, jnp.float32)
    tau_ref[...] = jnp.broadcast_to(tau, (R, 128))
    cgt_ref[...] = jnp.broadcast_to(cgt, (R, 128))


def _select(scores):
    return pl.pallas_call(
        _select_body,
        grid=(1,),
        in_specs=[pl.BlockSpec((R, KPAD), lambda i: (0, 0))],
        out_specs=[pl.BlockSpec((R, 128), lambda i: (0, 0)),
                   pl.BlockSpec((R, 128), lambda i: (0, 0))],
        out_shape=[jax.ShapeDtypeStruct((R, 128), jnp.float32),
                   jax.ShapeDtypeStruct((R, 128), jnp.int32)],
        interpret=_INTERP,
    )(scores)


NS = 16
SHARD = KPAD // NS
NCHUNK = SHARD // 16
CAND = 272


def _sc_retrieve(scores_flat, tau_flat, cgt_flat, mem):
    mesh = plsc.VectorSubcoreMesh(core_axis_name="c", subcore_axis_name="s")

    @functools.partial(
        pl.kernel, mesh=mesh,
        out_type=jax.ShapeDtypeStruct((R * TOPK, DIM), jnp.float32),
        scratch_types=[
            pltpu.VMEM((SHARD,), jnp.float32),
            pltpu.VMEM((16,), jnp.float32),
            pltpu.VMEM((CAND,), jnp.int32),
            pltpu.VMEM((CAND,), jnp.int32),
            pltpu.VMEM((16,), jnp.int32),
            pltpu.VMEM((NS * 16,), jnp.int32),
            pltpu.VMEM((NS * 16,), jnp.int32),
            pltpu.VMEM((CAND,), jnp.int32),
            pltpu.VMEM((TOPK,), jnp.int32),
            pltpu.VMEM((16,), jnp.int32),
            pltpu.VMEM((16, DIM), jnp.float32),
            pltpu.VMEM_SHARED((NS * CAND,), jnp.int32),
            pltpu.VMEM_SHARED((NS * CAND,), jnp.int32),
            pltpu.VMEM_SHARED((NS * 16,), jnp.int32),
            pltpu.VMEM_SHARED((NS * 16,), jnp.int32),
            pltpu.VMEM_SHARED((TOPK,), jnp.int32),
            pltpu.SemaphoreType.DMA,
        ],
        compiler_params=pltpu.CompilerParams(needs_layout_passes=False),
    )
    def body(scores_hbm, tau_hbm, cgt_hbm, mem_hbm, kv_hbm, sbuf, tauv,
             cand_gt, cand_eq, cntst, cntg_all, cntq_all, tmpbuf, idx_vmem,
             idx16, rowbuf, gt_sh, eq_sh, cntg_sh, cntq_sh, idx_sh, sem):
        del cgt_hbm
        cid = lax.axis_index("c")
        sid = lax.axis_index("s")
        iota = lax.iota(jnp.int32, 16)
        zero = jnp.zeros((16,), jnp.int32)
        one = jnp.ones((16,), jnp.int32)

        for r in range(R):
            core = r % 2

            @pl.when(cid == core)
            def _do_r(r=r):
                base = sid * SHARD
                pltpu.sync_copy(
                    scores_hbm.at[pl.ds(r * KPAD + base, SHARD)], sbuf)
                pltpu.sync_copy(tau_hbm.at[pl.ds(r * 128, 16)], tauv)
                tau = tauv[...]

                def scan_body(i, carry):
                    og, oq = carry
                    v = sbuf[pl.ds(i * 16, 16)]
                    gidx = base + i * 16 + iota
                    gt = v > tau
                    eq = v == tau
                    pos_g = og + plsc.cumsum(jnp.where(gt, one, zero)) - 1
                    gt2 = gt & (pos_g < TOPK)
                    plsc.store_scatter(cand_gt,
                                       [jnp.clip(pos_g, 0, CAND - 1)],
                                       gidx, mask=gt2)
                    pos_q = oq + plsc.cumsum(jnp.where(eq, one, zero)) - 1
                    eq2 = eq & (pos_q < TOPK)
                    plsc.store_scatter(cand_eq,
                                       [jnp.clip(pos_q, 0, CAND - 1)],
                                       gidx, mask=eq2)
                    return (og + plsc.all_reduce_population_count(gt2),
                            oq + plsc.all_reduce_population_count(eq2))

                og, oq = lax.fori_loop(0, NCHUNK, scan_body, (zero, zero))
                cntst[...] = og
                pltpu.sync_copy(cntst, cntg_sh.at[pl.ds(sid * 16, 16)])
                cntst[...] = oq
                pltpu.sync_copy(cntst, cntq_sh.at[pl.ds(sid * 16, 16)])
                pltpu.sync_copy(cand_gt, gt_sh.at[pl.ds(sid * CAND, CAND)])
                pltpu.sync_copy(cand_eq, eq_sh.at[pl.ds(sid * CAND, CAND)])
                plsc.subcore_barrier()

                @pl.when(sid == 0)
                def _assemble():
                    pltpu.sync_copy(cntg_sh, cntg_all)
                    pltpu.sync_copy(cntq_sh, cntq_all)

                    def merge(cnt_all, src_sh, off):
                        for j in range(NS):
                            cj = cnt_all[pl.ds(j * 16, 16)]
                            pltpu.sync_copy(
                                src_sh.at[pl.ds(j * CAND, CAND)], tmpbuf)

                            def abody(c, off, cj=cj):
                                v = tmpbuf[pl.ds(c * 16, 16)]
                                valid = (c * 16 + iota) < cj
                                pos = off + plsc.cumsum(
                                    jnp.where(valid, one, zero)) - 1
                                keep = valid & (pos < TOPK)
                                plsc.store_scatter(
                                    idx_vmem, [jnp.clip(pos, 0, TOPK - 1)],
                                    v, mask=keep)
                                return off + plsc.all_reduce_population_count(keep)

                            off = lax.fori_loop(0, CAND // 16, abody, off)
                        return off

                    off = merge(cntg_all, gt_sh, zero)
                    merge(cntq_all, eq_sh, off)
                    pltpu.sync_copy(idx_vmem, idx_sh)

                plsc.subcore_barrier()

                pltpu.sync_copy(idx_sh.at[pl.ds(sid * 16, 16)], idx16)
                idx16[...] = jnp.clip(idx16[...], 0, KMEM - 1)
                pltpu.async_copy(mem_hbm.at[idx16], rowbuf, sem).wait()
                pltpu.sync_copy(rowbuf,
                                kv_hbm.at[pl.ds(r * TOPK + sid * 16, 16)])

    return body(scores_flat, tau_flat, cgt_flat, mem)


def _qproj_body(x_ref, wq_ref, bq_ref, q_ref):
    bf = jnp.bfloat16
    q_ref[...] = (jnp.dot(x_ref[...].astype(bf), wq_ref[...].astype(bf),
                          preferred_element_type=jnp.float32)
                  + bq_ref[...]).astype(bf)


def _qproj(x, wq, bq):
    full = lambda shape: pl.BlockSpec(shape, lambda: tuple(0 for _ in shape))
    return pl.pallas_call(
        _qproj_body,
        in_specs=[full((S, DIM)), full((DIM, DIM)), full((1, DIM))],
        out_specs=full((S, DIM)),
        out_shape=jax.ShapeDtypeStruct((S, DIM), jnp.bfloat16),
        interpret=_INTERP,
    )(x, wq, bq)


def _fuse_body(x_ref, q_ref, kv_ref, wk_ref, bk_ref, wv_ref, bv_ref,
               wo_ref, bo_ref, g_ref, b_ref, out_ref):
    bf = jnp.bfloat16
    x = x_ref[...]
    kv = kv_ref[...].astype(bf)
    q = q_ref[...]
    k = (jnp.dot(kv, wk_ref[...].astype(bf),
                 preferred_element_type=jnp.float32) + bk_ref[...]).astype(bf)
    v = (jnp.dot(kv, wv_ref[...].astype(bf),
                 preferred_element_type=jnp.float32) + bv_ref[...]).astype(bf)
    scale = 1.0 / jnp.sqrt(jnp.float32(DH))
    outs = []
    for h in range(HEADS):
        qh = q[:, h * DH:(h + 1) * DH]
        kh = k[:, h * DH:(h + 1) * DH]
        vh = v[:, h * DH:(h + 1) * DH]
        lg = lax.dot_general(qh, kh, (((1,), (1,)), ((), ())),
                             preferred_element_type=jnp.float32) * scale
        m = jnp.max(lg, axis=1, keepdims=True)
        e = jnp.exp(lg - m)
        aw = (e / jnp.sum(e, axis=1, keepdims=True)).astype(jnp.bfloat16)
        outs.append(jnp.dot(aw, vh, preferred_element_type=jnp.float32))
    out = jnp.concatenate(outs, axis=1).astype(jnp.bfloat16)
    fused = jnp.dot(out, wo_ref[...].astype(jnp.bfloat16),
                    preferred_element_type=jnp.float32) + bo_ref[...]
    y = x + fused
    mu = jnp.mean(y, axis=1, keepdims=True)
    var = jnp.mean((y - mu) ** 2, axis=1, keepdims=True)
    out_ref[...] = (y - mu) / jnp.sqrt(var + 1e-5) * g_ref[...] + b_ref[...]


def _fuse(x, q, kv, wk, bk, wv, bv, wo, bo, g, b):
    full = lambda shape: pl.BlockSpec(shape, lambda: tuple(0 for _ in shape))
    return pl.pallas_call(
        _fuse_body,
        in_specs=[
            full((S, DIM)), full((S, DIM)), full((R * TOPK, DIM)),
            full((DIM, DIM)), full((1, DIM)),
            full((DIM, DIM)), full((1, DIM)),
            full((DIM, DIM)), full((1, DIM)),
            full((1, DIM)), full((1, DIM)),
        ],
        out_specs=full((S, DIM)),
        out_shape=jax.ShapeDtypeStruct((S, DIM), jnp.float32),
        interpret=_INTERP,
    )(x, q, kv, wk, bk, wv, bv, wo, bo, g, b)


def kernel(x, mem_tokens, g1_w, g1_b, g2_w, g2_b, pool_gamma, pool_beta,
           pool_w, pool_b, rel1_w, rel1_b, rel2_w, rel2_b, wq, bq, wk, bk,
           wv, bv, wo, bo, ln_gamma, ln_beta):
    x2 = x[0]
    row = lambda t: t.reshape(1, -1)
    queries = _queries(x2, g1_w, g1_b, g2_w, g2_b, pool_gamma, pool_beta,
                       pool_w, pool_b)
    scores = _scores(queries, mem_tokens)
    tau, cgt = _select(scores)
    kv = _sc_retrieve(scores.reshape(-1), tau.reshape(-1), cgt.reshape(-1),
                      mem_tokens)
    q = _qproj(x2, wq, row(bq))
    out = _fuse(x2, q, kv, wk, row(bk), wv, row(bv), wo, row(bo),
                row(ln_gamma), row(ln_beta))
    return out[None]
